# baseline (speedup 1.0000x reference)
import jax
import jax.numpy as jnp
from jax.experimental import pallas as pl

_LEN_KEEP = 512
_SEQ = 1024
_FEAT = 16
_BLK_ROWS = 64
_BIG = 1 << 24


_K1 = 928981903
_K2 = 3453687069
_KS2 = _K1 ^ _K2 ^ 0x1BD11BDA
_ROT = ((13, 15, 26, 6), (17, 29, 16, 24))


def _count_less(vals, cand):
    return jnp.sum((vals < cand).astype(jnp.int32), axis=-1, keepdims=True)


def _uniform_bits(flat_base, shape):
    rows, cols = shape
    n = (flat_base
         + jax.lax.broadcasted_iota(jnp.uint32, shape, 0) * jnp.uint32(cols)
         + jax.lax.broadcasted_iota(jnp.uint32, shape, 1))
    ks = (jnp.uint32(_K1), jnp.uint32(_K2), jnp.uint32(_KS2))
    x0 = jnp.full(shape, _K1, jnp.uint32)
    x1 = n + ks[1]
    for i in range(5):
        for r in _ROT[i % 2]:
            x0 = x0 + x1
            x1 = (x1 << r) | (x1 >> (32 - r))
            x1 = x1 ^ x0
        x0 = x0 + ks[(i + 1) % 3]
        x1 = x1 + ks[(i + 2) % 3] + jnp.uint32(i + 1)
    return ((x0 ^ x1) >> 9).astype(jnp.int32)


def _mask_apply_kernel(patch_ref, out_ref, mask_ref):
    r = _BLK_ROWS
    base = (pl.program_id(0) * (_BLK_ROWS * _SEQ)).astype(jnp.uint32)
    ui = _uniform_bits(base, (r, _SEQ))
    prefix = jnp.zeros((r, 1), jnp.int32)
    for shift in (20, 17, 14, 11):
        d = jnp.zeros((r, 1), jnp.int32)
        for k in range(1, 8):
            cnt = _count_less(ui, prefix | (k << shift))
            d = d + (cnt <= _LEN_KEEP).astype(jnp.int32)
        prefix = prefix | (d << shift)
    cnt_less = _count_less(ui, prefix)
    m = _LEN_KEEP - cnt_less
    grp = (ui >> 11) == (prefix >> 11)
    idx = jax.lax.broadcasted_iota(jnp.int32, ui.shape, 1)
    key2 = ((ui & 0x7FF) << 11) | idx
    kmask_lo = jnp.where(grp, key2, _BIG)
    kmask_hi = jnp.where(grp, key2, jnp.int32(-1))
    k0 = jnp.min(kmask_lo, axis=-1, keepdims=True)
    kz = jnp.max(kmask_hi, axis=-1, keepdims=True)
    k1 = jnp.min(jnp.where(kmask_lo > k0, kmask_lo, _BIG), axis=-1,
                 keepdims=True)
    kz1 = jnp.max(jnp.where(kmask_hi < kz, kmask_hi, -1), axis=-1,
                  keepdims=True)
    k2 = jnp.min(jnp.where((kmask_lo > k0) & (kmask_lo > k1), kmask_lo,
                           _BIG), axis=-1, keepdims=True)
    kz2 = jnp.max(jnp.where((kmask_hi < kz) & (kmask_hi < kz1), kmask_hi,
                            -1), axis=-1, keepdims=True)
    gcnt = jnp.sum(grp.astype(jnp.int32), axis=-1, keepdims=True)
    gsum = jnp.sum(jnp.where(grp, key2, 0), axis=-1, keepdims=True)
    kmid = gsum - k0 - k1 - k2 - kz - kz1 - kz2
    t2 = jnp.where(
        m == 0, k0,
        jnp.where(m == 1, k1,
                  jnp.where(m == 2, k2,
                            jnp.where(m == gcnt - 1, kz,
                                      jnp.where(m == gcnt - 2, kz1,
                                                jnp.where(m == gcnt - 3, kz2,
                                                          kmid))))))
    keep = (ui < prefix) | (grp & (key2 < t2))
    keep_f = keep.astype(jnp.float32)
    mask_ref[...] = 1.0 - keep_f
    ei = jax.lax.broadcasted_iota(jnp.int32, (_FEAT * r, r), 0)
    ej = jax.lax.broadcasted_iota(jnp.int32, (_FEAT * r, r), 1)
    expand = ((ei // _FEAT) == ej).astype(jnp.float32)
    keep16 = jnp.dot(expand, keep_f, preferred_element_type=jnp.float32)
    out_ref[...] = patch_ref[...] * keep16


def kernel(patch_input):
    b, c, s, f = patch_input.shape
    rows = b * c
    patch = patch_input.transpose(0, 1, 3, 2).reshape(rows * f, s)

    grid = (rows // _BLK_ROWS,)
    out, mask = pl.pallas_call(
        _mask_apply_kernel,
        grid=grid,
        in_specs=[
            pl.BlockSpec((_BLK_ROWS * f, s), lambda i: (i, 0)),
        ],
        out_specs=[
            pl.BlockSpec((_BLK_ROWS * f, s), lambda i: (i, 0)),
            pl.BlockSpec((_BLK_ROWS, s), lambda i: (i, 0)),
        ],
        out_shape=[
            jax.ShapeDtypeStruct((rows * f, s), jnp.float32),
            jax.ShapeDtypeStruct((rows, s), jnp.float32),
        ],
    )(patch)
    out4 = out.reshape(b, c, f, s).transpose(0, 1, 3, 2)
    return out4, mask.reshape(b, c, s)


# speedup vs baseline: 1.0315x; 1.0315x over previous
import jax
import jax.numpy as jnp
from jax.experimental import pallas as pl

_LEN_KEEP = 512
_SEQ = 1024
_FEAT = 16
_BLK_ROWS = 64
_BIG = 1 << 24


_K1 = 928981903
_K2 = 3453687069
_KS2 = _K1 ^ _K2 ^ 0x1BD11BDA
_ROT = ((13, 15, 26, 6), (17, 29, 16, 24))


def _count_less(vals, cand):
    return jnp.sum((vals < cand).astype(jnp.int32), axis=-1, keepdims=True)


def _uniform_bits(flat_base, shape):
    rows, cols = shape
    n = (flat_base
         + jax.lax.broadcasted_iota(jnp.uint32, shape, 0) * jnp.uint32(cols)
         + jax.lax.broadcasted_iota(jnp.uint32, shape, 1))
    ks = (jnp.uint32(_K1), jnp.uint32(_K2), jnp.uint32(_KS2))
    x0 = jnp.full(shape, _K1, jnp.uint32)
    x1 = n + ks[1]
    for i in range(5):
        for r in _ROT[i % 2]:
            x0 = x0 + x1
            x1 = (x1 << r) | (x1 >> (32 - r))
            x1 = x1 ^ x0
        x0 = x0 + ks[(i + 1) % 3]
        x1 = x1 + ks[(i + 2) % 3] + jnp.uint32(i + 1)
    return ((x0 ^ x1) >> 9).astype(jnp.int32)


def _mask_apply_kernel(patch_ref, out_ref, mask_ref):
    r = _BLK_ROWS
    base = (pl.program_id(0) * (_BLK_ROWS * _SEQ)).astype(jnp.uint32)
    ui = _uniform_bits(base, (r, _SEQ))
    prefix = jnp.zeros((r, 1), jnp.int32)
    cnt_lo = jnp.zeros((r, 1), jnp.int32)
    cnt_hi = jnp.full((r, 1), _SEQ, jnp.int32)
    for shift in (21, 19, 17, 15, 13, 11):
        c1 = _count_less(ui, prefix | (1 << shift))
        c2 = _count_less(ui, prefix | (2 << shift))
        c3 = _count_less(ui, prefix | (3 << shift))
        b1 = c1 <= _LEN_KEEP
        b2 = c2 <= _LEN_KEEP
        b3 = c3 <= _LEN_KEEP
        d = (b1.astype(jnp.int32) + b2.astype(jnp.int32)
             + b3.astype(jnp.int32))
        prefix = prefix | (d << shift)
        cnt_lo = jnp.where(b1, jnp.where(b2, jnp.where(b3, c3, c2), c1),
                           cnt_lo)
        cnt_hi = jnp.where(b1, jnp.where(b2, jnp.where(b3, cnt_hi, c3), c2),
                           c1)
    m = _LEN_KEEP - cnt_lo
    gcnt = cnt_hi - cnt_lo
    grp = (ui >> 11) == (prefix >> 11)
    idx = jax.lax.broadcasted_iota(jnp.int32, ui.shape, 1)
    key2 = ((ui & 0x7FF) << 11) | idx
    kmask_lo = jnp.where(grp, key2, _BIG)
    kmask_hi = jnp.where(grp, key2, jnp.int32(-1))
    k0 = jnp.min(kmask_lo, axis=-1, keepdims=True)
    kz = jnp.max(kmask_hi, axis=-1, keepdims=True)
    k1 = jnp.min(jnp.where(kmask_lo > k0, kmask_lo, _BIG), axis=-1,
                 keepdims=True)
    kz1 = jnp.max(jnp.where(kmask_hi < kz, kmask_hi, -1), axis=-1,
                  keepdims=True)
    k2 = jnp.min(jnp.where((kmask_lo > k0) & (kmask_lo > k1), kmask_lo,
                           _BIG), axis=-1, keepdims=True)
    kz2 = jnp.max(jnp.where((kmask_hi < kz) & (kmask_hi < kz1), kmask_hi,
                            -1), axis=-1, keepdims=True)
    gsum = jnp.sum(jnp.where(grp, key2, 0), axis=-1, keepdims=True)
    kmid = gsum - k0 - k1 - k2 - kz - kz1 - kz2
    t2 = jnp.where(
        m == 0, k0,
        jnp.where(m == 1, k1,
                  jnp.where(m == 2, k2,
                            jnp.where(m == gcnt - 1, kz,
                                      jnp.where(m == gcnt - 2, kz1,
                                                jnp.where(m == gcnt - 3, kz2,
                                                          kmid))))))
    keep = (ui < prefix) | (grp & (key2 < t2))
    keep_f = keep.astype(jnp.float32)
    mask_ref[...] = 1.0 - keep_f
    ei = jax.lax.broadcasted_iota(jnp.int32, (_FEAT * r, r), 0)
    ej = jax.lax.broadcasted_iota(jnp.int32, (_FEAT * r, r), 1)
    expand = ((ei // _FEAT) == ej).astype(jnp.float32)
    keep16 = jnp.dot(expand, keep_f, preferred_element_type=jnp.float32)
    out_ref[...] = patch_ref[...] * keep16


def kernel(patch_input):
    b, c, s, f = patch_input.shape
    rows = b * c
    patch = patch_input.transpose(0, 1, 3, 2).reshape(rows * f, s)

    grid = (rows // _BLK_ROWS,)
    out, mask = pl.pallas_call(
        _mask_apply_kernel,
        grid=grid,
        in_specs=[
            pl.BlockSpec((_BLK_ROWS * f, s), lambda i: (i, 0)),
        ],
        out_specs=[
            pl.BlockSpec((_BLK_ROWS * f, s), lambda i: (i, 0)),
            pl.BlockSpec((_BLK_ROWS, s), lambda i: (i, 0)),
        ],
        out_shape=[
            jax.ShapeDtypeStruct((rows * f, s), jnp.float32),
            jax.ShapeDtypeStruct((rows, s), jnp.float32),
        ],
    )(patch)
    out4 = out.reshape(b, c, f, s).transpose(0, 1, 3, 2)
    return out4, mask.reshape(b, c, s)


# speedup vs baseline: 1.1481x; 1.1130x over previous
import jax
import jax.numpy as jnp
from jax.experimental import pallas as pl

_LEN_KEEP = 512
_SEQ = 1024
_FEAT = 16
_BLK_ROWS = 128
_BIG = 1 << 24


_K1 = 928981903
_K2 = 3453687069
_KS2 = _K1 ^ _K2 ^ 0x1BD11BDA
_ROT = ((13, 15, 26, 6), (17, 29, 16, 24))


def _count_less(vals, cand):
    return jnp.sum((vals < cand).astype(jnp.int32), axis=-1, keepdims=True)


def _uniform_bits(flat_base, shape):
    rows, cols = shape
    n = (flat_base
         + jax.lax.broadcasted_iota(jnp.uint32, shape, 0) * jnp.uint32(cols)
         + jax.lax.broadcasted_iota(jnp.uint32, shape, 1))
    ks = (jnp.uint32(_K1), jnp.uint32(_K2), jnp.uint32(_KS2))
    x0 = jnp.full(shape, _K1, jnp.uint32)
    x1 = n + ks[1]
    for i in range(5):
        for r in _ROT[i % 2]:
            x0 = x0 + x1
            x1 = (x1 << r) | (x1 >> (32 - r))
            x1 = x1 ^ x0
        x0 = x0 + ks[(i + 1) % 3]
        x1 = x1 + ks[(i + 2) % 3] + jnp.uint32(i + 1)
    return ((x0 ^ x1) >> 9).astype(jnp.int32)


def _mask_apply_kernel(patch_ref, out_ref, mask_ref):
    r = _BLK_ROWS
    base = (pl.program_id(0) * (_BLK_ROWS * _SEQ)).astype(jnp.uint32)
    ui = _uniform_bits(base, (r, _SEQ))
    prefix = jnp.zeros((r, 1), jnp.int32)
    cnt_lo = jnp.zeros((r, 1), jnp.int32)
    cnt_hi = jnp.full((r, 1), _SEQ, jnp.int32)
    for shift in (21, 19, 17, 15, 13, 11):
        c1 = _count_less(ui, prefix | (1 << shift))
        c2 = _count_less(ui, prefix | (2 << shift))
        c3 = _count_less(ui, prefix | (3 << shift))
        b1 = c1 <= _LEN_KEEP
        b2 = c2 <= _LEN_KEEP
        b3 = c3 <= _LEN_KEEP
        d = (b1.astype(jnp.int32) + b2.astype(jnp.int32)
             + b3.astype(jnp.int32))
        prefix = prefix | (d << shift)
        cnt_lo = jnp.where(b1, jnp.where(b2, jnp.where(b3, c3, c2), c1),
                           cnt_lo)
        cnt_hi = jnp.where(b1, jnp.where(b2, jnp.where(b3, cnt_hi, c3), c2),
                           c1)
    m = _LEN_KEEP - cnt_lo
    gcnt = cnt_hi - cnt_lo
    grp = (ui >> 11) == (prefix >> 11)
    idx = jax.lax.broadcasted_iota(jnp.int32, ui.shape, 1)
    key2 = ((ui & 0x7FF) << 11) | idx
    kmask_lo = jnp.where(grp, key2, _BIG)
    kmask_hi = jnp.where(grp, key2, jnp.int32(-1))
    k0 = jnp.min(kmask_lo, axis=-1, keepdims=True)
    kz = jnp.max(kmask_hi, axis=-1, keepdims=True)
    k1 = jnp.min(jnp.where(kmask_lo > k0, kmask_lo, _BIG), axis=-1,
                 keepdims=True)
    kz1 = jnp.max(jnp.where(kmask_hi < kz, kmask_hi, -1), axis=-1,
                  keepdims=True)
    k2 = jnp.min(jnp.where((kmask_lo > k0) & (kmask_lo > k1), kmask_lo,
                           _BIG), axis=-1, keepdims=True)
    kz2 = jnp.max(jnp.where((kmask_hi < kz) & (kmask_hi < kz1), kmask_hi,
                            -1), axis=-1, keepdims=True)
    gsum = jnp.sum(jnp.where(grp, key2, 0), axis=-1, keepdims=True)
    kmid = gsum - k0 - k1 - k2 - kz - kz1 - kz2
    t2 = jnp.where(
        m == 0, k0,
        jnp.where(m == 1, k1,
                  jnp.where(m == 2, k2,
                            jnp.where(m == gcnt - 1, kz,
                                      jnp.where(m == gcnt - 2, kz1,
                                                jnp.where(m == gcnt - 3, kz2,
                                                          kmid))))))
    keep = (ui < prefix) | (grp & (key2 < t2))
    keep_f = keep.astype(jnp.float32)
    mask_ref[...] = 1.0 - keep_f
    ei = jax.lax.broadcasted_iota(jnp.int32, (_FEAT * r, r), 0)
    ej = jax.lax.broadcasted_iota(jnp.int32, (_FEAT * r, r), 1)
    expand = ((ei // _FEAT) == ej).astype(jnp.float32)
    keep16 = jnp.dot(expand, keep_f, preferred_element_type=jnp.float32)
    out_ref[...] = patch_ref[...] * keep16


def kernel(patch_input):
    b, c, s, f = patch_input.shape
    rows = b * c
    patch = patch_input.transpose(0, 1, 3, 2).reshape(rows * f, s)

    grid = (rows // _BLK_ROWS,)
    out, mask = pl.pallas_call(
        _mask_apply_kernel,
        grid=grid,
        in_specs=[
            pl.BlockSpec((_BLK_ROWS * f, s), lambda i: (i, 0)),
        ],
        out_specs=[
            pl.BlockSpec((_BLK_ROWS * f, s), lambda i: (i, 0)),
            pl.BlockSpec((_BLK_ROWS, s), lambda i: (i, 0)),
        ],
        out_shape=[
            jax.ShapeDtypeStruct((rows * f, s), jnp.float32),
            jax.ShapeDtypeStruct((rows, s), jnp.float32),
        ],
    )(patch)
    out4 = out.reshape(b, c, f, s).transpose(0, 1, 3, 2)
    return out4, mask.reshape(b, c, s)
